# baseline (speedup 1.0000x reference)
import functools

import jax
import jax.numpy as jnp
from jax import lax
from jax.experimental import pallas as pl
from jax.experimental.pallas import tpu as pltpu
from jax.experimental.pallas import tpu_sc as plsc

N = 10000
E = 320000
NC, NS, L = 2, 16, 16
NW = NC * NS
EW = E // NW
B = 64
NCHUNK = EW // B
TAIL = EW - NCHUNK * B
NG = B // L
RPT = N // NS


@functools.lru_cache(maxsize=None)
def _edge_phase(DM, CD):
    UN = 4 if CD % 4 == 0 else 1
    mesh = plsc.VectorSubcoreMesh(core_axis_name="c", subcore_axis_name="s",
                                  num_cores=NC, num_subcores=NS)

    @functools.partial(
        pl.kernel,
        mesh=mesh,
        compiler_params=pltpu.CompilerParams(use_tc_tiling_on_sc=False,
                                             needs_layout_passes=False),
        out_type=jax.ShapeDtypeStruct((NC, N, DM), jnp.float32),
        scratch_types=[
            pltpu.VMEM((4, B), jnp.int32),
            pltpu.VMEM((4, B), jnp.int32),
            pltpu.VMEM((B, DM), jnp.float32),
            pltpu.VMEM((B, DM), jnp.float32),
            pltpu.VMEM((B, DM), jnp.float32),
            pltpu.VMEM((B, DM), jnp.float32),
            pltpu.VMEM((CD, L), jnp.float32),
            pltpu.VMEM((TAIL,), jnp.int32),
            pltpu.VMEM((TAIL,), jnp.int32),
            pltpu.VMEM_SHARED((N, DM), jnp.float32),
            pltpu.SemaphoreType.DMA,
            pltpu.SemaphoreType.DMA,
            pltpu.SemaphoreType.DMA,
            pltpu.SemaphoreType.DMA,
            pltpu.SemaphoreType.DMA,
            pltpu.SemaphoreType.DMA,
            pltpu.SemaphoreType.DMA,
            pltpu.SemaphoreType.DMA,
        ],
    )
    def edge_kernel(xl_hbm, xr_hbm, src_hbm, dst_hbm, att_hbm, zer_hbm,
                    acc_out,
                    src_i, dst_i, xl0, xl1, xr0, xr1, att_v, src_t, dst_t,
                    acc_sh, i0, i1, i2, i3, g0, g1, s0, s1):
        cid = lax.axis_index("c")
        sid = lax.axis_index("s")
        wid = sid * NC + cid
        xl_r, xr_r = [xl0, xl1], [xr0, xr1]
        isem, gsem, ssem = [i0, i1, i2, i3], [g0, g1], [s0, s1]

        row0 = sid * RPT
        pltpu.sync_copy(zer_hbm, acc_sh.at[pl.ds(row0, RPT), :])
        pltpu.sync_copy(att_hbm, att_v)
        plsc.subcore_barrier()

        base = wid * EW
        last = NCHUNK - 1
        iota = lax.iota(jnp.int32, L)
        lincol = jnp.full((L,), CD, jnp.int32)
        zero16 = jnp.zeros((L,), jnp.float32)

        def idx_fetch(chunk, slot, sem):
            off = base + chunk * B
            pltpu.make_async_copy(src_hbm.at[pl.ds(off, B)],
                                  src_i.at[slot], sem).start()
            pltpu.make_async_copy(dst_hbm.at[pl.ds(off, B)],
                                  dst_i.at[slot], sem).start()

        def idx_wait(slot, sem):
            pltpu.make_async_copy(src_hbm.at[pl.ds(base, B)],
                                  src_i.at[slot], sem).wait()
            pltpu.make_async_copy(dst_hbm.at[pl.ds(base, B)],
                                  dst_i.at[slot], sem).wait()

        def gather_start(slot, p):
            pltpu.make_async_copy(xl_hbm.at[src_i.at[slot]],
                                  xl_r[p], gsem[p]).start()
            pltpu.make_async_copy(xr_hbm.at[dst_i.at[slot]],
                                  xr_r[p], gsem[p]).start()

        def gather_wait(slot, p):
            pltpu.make_async_copy(xl_hbm.at[src_i.at[slot]],
                                  xl_r[p], gsem[p]).wait()
            pltpu.make_async_copy(xr_hbm.at[dst_i.at[slot]],
                                  xr_r[p], gsem[p]).wait()

        def scatter_start(slot, p):
            pltpu.make_async_copy(xl_r[p], acc_sh.at[dst_i.at[slot]],
                                  ssem[p]).start(add=True)

        def scatter_wait(p):
            pltpu.make_async_copy(xl_r[p], acc_sh.at[dst_i.at[0]],
                                  ssem[p]).wait()

        def compute(xlb, xrb, ng):
            rows = [iota + g * L for g in range(ng)]

            def chan(c, accs):
                for u in range(UN):
                    cc = c * UN + u
                    av = att_v[cc, :]
                    colv = jnp.full((L,), cc, jnp.int32)
                    for g in range(ng):
                        a = plsc.load_gather(xlb, [rows[g], colv])
                        b = plsc.load_gather(xrb, [rows[g], colv])
                        z = jnp.maximum(a + b, 0.0)
                        accs = accs[:g] + (accs[g] + av * z,) + accs[g + 1:]
                return accs
            accs = lax.fori_loop(0, CD // UN, chan,
                                 tuple(zero16 for _ in range(ng)))

            wvs = []
            for g in range(ng):
                la = plsc.load_gather(xlb, [rows[g], lincol])
                lb = plsc.load_gather(xrb, [rows[g], lincol])
                wv = jnp.exp(la + lb + accs[g])
                plsc.store_scatter(xlb, [rows[g], lincol], wv)
                wvs.append(wv)

            def scale(c, _):
                for u in range(UN):
                    cc = c * UN + u
                    colv = jnp.full((L,), cc, jnp.int32)
                    for g in range(ng):
                        a = plsc.load_gather(xlb, [rows[g], colv])
                        plsc.store_scatter(xlb, [rows[g], colv], a * wvs[g])
                return 0
            lax.fori_loop(0, CD // UN, scale, 0)

        def process(c, s_idx):
            p = s_idx % 2
            s_nxt, s_pre = (s_idx + 1) % 4, (s_idx + 2) % 4
            p_nxt = (p + 1) % 2
            gather_wait(s_idx, p)
            idx_fetch(jnp.minimum(c + 2, last), s_pre, isem[s_pre])

            @pl.when(c >= 1)
            def _():
                scatter_wait(p_nxt)
            idx_wait(s_nxt, isem[s_nxt])
            gather_start(s_nxt, p_nxt)
            compute(xl_r[p], xr_r[p], NG)
            scatter_start(s_idx, p)

        pltpu.sync_copy(src_hbm.at[pl.ds(base, B)], src_i.at[0])
        pltpu.sync_copy(dst_hbm.at[pl.ds(base, B)], dst_i.at[0])
        idx_fetch(1, 1, isem[1])
        gather_start(0, 0)

        def quad(i, _):
            for c4 in range(4):
                process(4 * i + c4, c4)
            return 0
        lax.fori_loop(0, NCHUNK // 4, quad, 0)
        for c in range(4 * (NCHUNK // 4), NCHUNK):
            process(c, c % 4)

        idx_wait((last + 2) % 4, isem[(last + 2) % 4])
        gather_wait((last + 1) % 4, (last + 1) % 2)
        scatter_wait(last % 2)

        toff = base + NCHUNK * B
        pltpu.sync_copy(src_hbm.at[pl.ds(toff, TAIL)], src_t)
        pltpu.sync_copy(dst_hbm.at[pl.ds(toff, TAIL)], dst_t)
        pltpu.sync_copy(xl_hbm.at[src_t], xl0.at[pl.ds(0, TAIL), :])
        pltpu.sync_copy(xr_hbm.at[dst_t], xr0.at[pl.ds(0, TAIL), :])
        compute(xl0, xr0, TAIL // L)
        pltpu.sync_copy(xl0.at[pl.ds(0, TAIL), :], acc_sh.at[dst_t],
                        add=True)

        plsc.subcore_barrier()
        pltpu.sync_copy(acc_sh.at[pl.ds(row0, RPT), :],
                        acc_out.at[cid, pl.ds(row0, RPT), :])

    return edge_kernel


def _matmul(x, w):
    R = 1000
    K, M = w.shape

    def body(x_ref, w_ref, o_ref):
        o_ref[...] = jnp.dot(x_ref[...], w_ref[...],
                             preferred_element_type=jnp.float32)

    return pl.pallas_call(
        body,
        grid=(N // R,),
        in_specs=[pl.BlockSpec((R, K), lambda i: (i, 0)),
                  pl.BlockSpec((K, M), lambda i: (0, 0))],
        out_specs=pl.BlockSpec((R, M), lambda i: (i, 0)),
        out_shape=jax.ShapeDtypeStruct((N, M), jnp.float32),
    )(x, w)


def _combine_matmul(acc, skip, bias, w):
    R = 1000
    D = acc.shape[2] - 16
    M = w.shape[1]

    def body(a_ref, s_ref, b_ref, w_ref, o_ref):
        raw = a_ref[0] + a_ref[1]
        ns = raw[:, :D]
        den = raw[:, D:D + 1]
        h = ns / (den + 1e-16) + s_ref[...] + b_ref[...]
        h = jnp.maximum(h, 0.0)
        o_ref[...] = jnp.dot(h, w_ref[...],
                             preferred_element_type=jnp.float32)

    return pl.pallas_call(
        body,
        grid=(N // R,),
        in_specs=[pl.BlockSpec((NC, R, D + 16), lambda i: (0, i, 0)),
                  pl.BlockSpec((R, D), lambda i: (i, 0)),
                  pl.BlockSpec((1, D), lambda i: (0, 0)),
                  pl.BlockSpec((D, M), lambda i: (0, 0))],
        out_specs=pl.BlockSpec((R, M), lambda i: (i, 0)),
        out_shape=jax.ShapeDtypeStruct((N, M), jnp.float32),
    )(acc, skip, bias, w)


def _final(acc, skip, bias):
    R = 1000

    def body(a_ref, s_ref, b_ref, o_ref):
        raw = a_ref[0] + a_ref[1]
        num = raw[:, 0:1]
        den = raw[:, 1:2]
        o_ref[...] = num / (den + 1e-16) + s_ref[:, 0:1] + b_ref[...]

    return pl.pallas_call(
        body,
        grid=(N // R,),
        in_specs=[pl.BlockSpec((NC, R, 32), lambda i: (0, i, 0)),
                  pl.BlockSpec((R, 16), lambda i: (i, 0)),
                  pl.BlockSpec((1, 16), lambda i: (0, 0))],
        out_specs=pl.BlockSpec((R, 16), lambda i: (i, 0)),
        out_shape=jax.ShapeDtypeStruct((N, 16), jnp.float32),
    )(acc, skip, bias)


def kernel(x, edge_index, Wl1, Wr1, att1, b1, Wlin1, blin1,
           Wl2, Wr2, att2, b2, Wlin2, blin2,
           Wlo, Wro, atto, bo, Wlino, blino):
    src = edge_index[0]
    dst = edge_index[1]

    zer144 = jnp.zeros((RPT, 144), jnp.float32)
    z15 = jnp.zeros((128, 15), jnp.float32)

    def wcat(Wl, Wr, att, Wlin):
        al = 0.2 * (Wl @ att)[:, None]
        ar = 0.2 * (Wr @ att)[:, None]
        return jnp.concatenate([Wl, al, z15, Wr, ar, z15, Wlin], axis=1)

    def aspl(att):
        return jnp.broadcast_to((0.8 * att)[:, None], (att.shape[0], 16))

    X1 = _matmul(x, wcat(Wl1, Wr1, att1, Wlin1))
    acc1 = _edge_phase(144, 128)(X1[:, :144], X1[:, 144:288], src, dst,
                                 aspl(att1), zer144)

    X2 = _combine_matmul(acc1, X1[:, 288:], (b1 + blin1)[None, :],
                         wcat(Wl2, Wr2, att2, Wlin2))
    acc2 = _edge_phase(144, 128)(X2[:, :144], X2[:, 144:288], src, dst,
                                 aspl(att2), zer144)

    z30 = jnp.zeros((128, 30), jnp.float32)
    W3 = jnp.concatenate([Wlo, 0.2 * atto[0] * Wlo, z30,
                          Wro, 0.2 * atto[0] * Wro, z30,
                          Wlino, jnp.zeros((128, 15), jnp.float32)], axis=1)
    X3 = _combine_matmul(acc2, X2[:, 288:], (b2 + blin2)[None, :], W3)

    acc3 = _edge_phase(32, 1)(X3[:, :32], X3[:, 32:64], src, dst,
                              aspl(atto), jnp.zeros((RPT, 32), jnp.float32))

    bc = jnp.broadcast_to((bo + blino)[None, :], (1, 16))
    out16 = _final(acc3, X3[:, 64:80], bc)
    return out16[:, 0:1]


# speedup vs baseline: 3.3407x; 3.3407x over previous
import functools

import jax
import jax.numpy as jnp
from jax import lax
from jax.experimental import pallas as pl
from jax.experimental.pallas import tpu as pltpu
from jax.experimental.pallas import tpu_sc as plsc

N = 10000
E = 320000
NC, NS, L = 2, 16, 16
NW = NC * NS
EW = E // NW
B = 64
NCHUNK = EW // B
TAIL = EW - NCHUNK * B
NG = B // L
RPT = N // NS


@functools.lru_cache(maxsize=None)
def _edge_phase(KG):
    D = KG * L
    DM = D + L
    mesh = plsc.VectorSubcoreMesh(core_axis_name="c", subcore_axis_name="s",
                                  num_cores=NC, num_subcores=NS)

    @functools.partial(
        pl.kernel,
        mesh=mesh,
        compiler_params=pltpu.CompilerParams(use_tc_tiling_on_sc=False,
                                             needs_layout_passes=False),
        out_type=jax.ShapeDtypeStruct((NC, N, DM), jnp.float32),
        scratch_types=[
            pltpu.VMEM((4, B), jnp.int32),
            pltpu.VMEM((4, B), jnp.int32),
            pltpu.VMEM((B, DM), jnp.float32),
            pltpu.VMEM((B, DM), jnp.float32),
            pltpu.VMEM((B, DM), jnp.float32),
            pltpu.VMEM((B, DM), jnp.float32),
            pltpu.VMEM((D,), jnp.float32),
            pltpu.VMEM((TAIL,), jnp.int32),
            pltpu.VMEM((TAIL,), jnp.int32),
            pltpu.VMEM_SHARED((N, DM), jnp.float32),
            pltpu.SemaphoreType.DMA,
            pltpu.SemaphoreType.DMA,
            pltpu.SemaphoreType.DMA,
            pltpu.SemaphoreType.DMA,
            pltpu.SemaphoreType.DMA,
            pltpu.SemaphoreType.DMA,
            pltpu.SemaphoreType.DMA,
            pltpu.SemaphoreType.DMA,
        ],
    )
    def edge_kernel(xl_hbm, xr_hbm, src_hbm, dst_hbm, att_hbm, zer_hbm,
                    acc_out,
                    src_i, dst_i, xl0, xl1, xr0, xr1, att_v, src_t, dst_t,
                    acc_sh, i0, i1, i2, i3, g0, g1, s0, s1):
        cid = lax.axis_index("c")
        sid = lax.axis_index("s")
        wid = sid * NC + cid
        xl_r, xr_r = [xl0, xl1], [xr0, xr1]
        isem, gsem, ssem = [i0, i1, i2, i3], [g0, g1], [s0, s1]

        row0 = sid * RPT
        pltpu.sync_copy(zer_hbm, acc_sh.at[pl.ds(row0, RPT), :])
        pltpu.sync_copy(att_hbm, att_v)
        plsc.subcore_barrier()

        base = wid * EW
        last = NCHUNK - 1
        att_k = [att_v[pl.ds(k * L, L)] for k in range(KG)]
        lane0 = (lax.iota(jnp.int32, L) == 0).astype(jnp.float32)

        def idx_fetch(chunk, slot, sem):
            off = base + chunk * B
            pltpu.make_async_copy(src_hbm.at[pl.ds(off, B)],
                                  src_i.at[slot], sem).start()
            pltpu.make_async_copy(dst_hbm.at[pl.ds(off, B)],
                                  dst_i.at[slot], sem).start()

        def idx_wait(slot, sem):
            pltpu.make_async_copy(src_hbm.at[pl.ds(base, B)],
                                  src_i.at[slot], sem).wait()
            pltpu.make_async_copy(dst_hbm.at[pl.ds(base, B)],
                                  dst_i.at[slot], sem).wait()

        def gather_start(slot, p):
            pltpu.make_async_copy(xl_hbm.at[src_i.at[slot]],
                                  xl_r[p], gsem[p]).start()
            pltpu.make_async_copy(xr_hbm.at[dst_i.at[slot]],
                                  xr_r[p], gsem[p]).start()

        def gather_wait(slot, p):
            pltpu.make_async_copy(xl_hbm.at[src_i.at[slot]],
                                  xl_r[p], gsem[p]).wait()
            pltpu.make_async_copy(xr_hbm.at[dst_i.at[slot]],
                                  xr_r[p], gsem[p]).wait()

        def scatter_start(slot, p):
            pltpu.make_async_copy(xl_r[p], acc_sh.at[dst_i.at[slot]],
                                  ssem[p]).start(add=True)

        def scatter_wait(p):
            pltpu.make_async_copy(xl_r[p], acc_sh.at[dst_i.at[0]],
                                  ssem[p]).wait()

        def compute(xlb, xrb, ne):
            def one(e):
                acc = None
                xls = []
                for k in range(KG):
                    a = xlb[e, pl.ds(k * L, L)]
                    b = xrb[e, pl.ds(k * L, L)]
                    t = jnp.maximum(a + b, 0.0) * att_k[k]
                    acc = t if acc is None else acc + t
                    xls.append(a)
                acc = acc + xlb[e, pl.ds(D, L)] + xrb[e, pl.ds(D, L)]
                wv = jnp.exp(jnp.full((L,), jnp.sum(acc), jnp.float32))
                xlb[e, pl.ds(D, L)] = wv * lane0
                for k in range(KG):
                    xlb[e, pl.ds(k * L, L)] = xls[k] * wv

            def four(j, _):
                for q in range(4):
                    one(4 * j + q)
                return 0
            lax.fori_loop(0, ne // 4, four, 0)

        def process(c, s_idx):
            p = s_idx % 2
            s_nxt, s_pre = (s_idx + 1) % 4, (s_idx + 2) % 4
            p_nxt = (p + 1) % 2
            gather_wait(s_idx, p)
            idx_fetch(jnp.minimum(c + 2, last), s_pre, isem[s_pre])

            @pl.when(c >= 1)
            def _():
                scatter_wait(p_nxt)
            idx_wait(s_nxt, isem[s_nxt])
            gather_start(s_nxt, p_nxt)
            compute(xl_r[p], xr_r[p], B)
            scatter_start(s_idx, p)

        pltpu.sync_copy(src_hbm.at[pl.ds(base, B)], src_i.at[0])
        pltpu.sync_copy(dst_hbm.at[pl.ds(base, B)], dst_i.at[0])
        idx_fetch(1, 1, isem[1])
        gather_start(0, 0)

        def quad(i, _):
            for c4 in range(4):
                process(4 * i + c4, c4)
            return 0
        lax.fori_loop(0, NCHUNK // 4, quad, 0)
        for c in range(4 * (NCHUNK // 4), NCHUNK):
            process(c, c % 4)

        idx_wait((last + 2) % 4, isem[(last + 2) % 4])
        gather_wait((last + 1) % 4, (last + 1) % 2)
        scatter_wait(last % 2)

        toff = base + NCHUNK * B
        pltpu.sync_copy(src_hbm.at[pl.ds(toff, TAIL)], src_t)
        pltpu.sync_copy(dst_hbm.at[pl.ds(toff, TAIL)], dst_t)
        pltpu.sync_copy(xl_hbm.at[src_t], xl0.at[pl.ds(0, TAIL), :])
        pltpu.sync_copy(xr_hbm.at[dst_t], xr0.at[pl.ds(0, TAIL), :])
        compute(xl0, xr0, TAIL)
        pltpu.sync_copy(xl0.at[pl.ds(0, TAIL), :], acc_sh.at[dst_t],
                        add=True)

        plsc.subcore_barrier()
        pltpu.sync_copy(acc_sh.at[pl.ds(row0, RPT), :],
                        acc_out.at[cid, pl.ds(row0, RPT), :])

    return edge_kernel


def _matmul(x, w):
    R = 1000
    K, M = w.shape

    def body(x_ref, w_ref, o_ref):
        o_ref[...] = jnp.dot(x_ref[...], w_ref[...],
                             preferred_element_type=jnp.float32)

    return pl.pallas_call(
        body,
        grid=(N // R,),
        in_specs=[pl.BlockSpec((R, K), lambda i: (i, 0)),
                  pl.BlockSpec((K, M), lambda i: (0, 0))],
        out_specs=pl.BlockSpec((R, M), lambda i: (i, 0)),
        out_shape=jax.ShapeDtypeStruct((N, M), jnp.float32),
    )(x, w)


def _combine_matmul(acc, skip, bias, w):
    R = 1000
    D = acc.shape[2] - 16
    M = w.shape[1]

    def body(a_ref, s_ref, b_ref, w_ref, o_ref):
        raw = a_ref[0] + a_ref[1]
        ns = raw[:, :D]
        den = raw[:, D:D + 1]
        h = ns / (den + 1e-16) + s_ref[...] + b_ref[...]
        h = jnp.maximum(h, 0.0)
        o_ref[...] = jnp.dot(h, w_ref[...],
                             preferred_element_type=jnp.float32)

    return pl.pallas_call(
        body,
        grid=(N // R,),
        in_specs=[pl.BlockSpec((NC, R, D + 16), lambda i: (0, i, 0)),
                  pl.BlockSpec((R, D), lambda i: (i, 0)),
                  pl.BlockSpec((1, D), lambda i: (0, 0)),
                  pl.BlockSpec((D, M), lambda i: (0, 0))],
        out_specs=pl.BlockSpec((R, M), lambda i: (i, 0)),
        out_shape=jax.ShapeDtypeStruct((N, M), jnp.float32),
    )(acc, skip, bias, w)


def _final(acc, skip, bias):
    R = 1000

    def body(a_ref, s_ref, b_ref, o_ref):
        raw = a_ref[0] + a_ref[1]
        num = raw[:, 0:1]
        den = raw[:, 16:17]
        o_ref[...] = num / (den + 1e-16) + s_ref[:, 0:1] + b_ref[...]

    return pl.pallas_call(
        body,
        grid=(N // R,),
        in_specs=[pl.BlockSpec((NC, R, 32), lambda i: (0, i, 0)),
                  pl.BlockSpec((R, 16), lambda i: (i, 0)),
                  pl.BlockSpec((1, 16), lambda i: (0, 0))],
        out_specs=pl.BlockSpec((R, 16), lambda i: (i, 0)),
        out_shape=jax.ShapeDtypeStruct((N, 16), jnp.float32),
    )(acc, skip, bias)


def kernel(x, edge_index, Wl1, Wr1, att1, b1, Wlin1, blin1,
           Wl2, Wr2, att2, b2, Wlin2, blin2,
           Wlo, Wro, atto, bo, Wlino, blino):
    src = edge_index[0]
    dst = edge_index[1]

    zer144 = jnp.zeros((RPT, 144), jnp.float32)
    z15 = jnp.zeros((128, 15), jnp.float32)

    def wcat(Wl, Wr, att, Wlin):
        al = 0.2 * (Wl @ att)[:, None]
        ar = 0.2 * (Wr @ att)[:, None]
        return jnp.concatenate([Wl, al, z15, Wr, ar, z15, Wlin], axis=1)

    X1 = _matmul(x, wcat(Wl1, Wr1, att1, Wlin1))
    acc1 = _edge_phase(8)(X1[:, :144], X1[:, 144:288], src, dst,
                          0.8 * att1, zer144)

    X2 = _combine_matmul(acc1, X1[:, 288:], (b1 + blin1)[None, :],
                         wcat(Wl2, Wr2, att2, Wlin2))
    acc2 = _edge_phase(8)(X2[:, :144], X2[:, 144:288], src, dst,
                          0.8 * att2, zer144)

    zc15 = jnp.zeros((128, 15), jnp.float32)
    W3 = jnp.concatenate([Wlo, zc15, 0.2 * atto[0] * Wlo, zc15,
                          Wro, zc15, 0.2 * atto[0] * Wro, zc15,
                          Wlino, zc15], axis=1)
    X3 = _combine_matmul(acc2, X2[:, 288:], (b2 + blin2)[None, :], W3)

    att3 = jnp.concatenate([0.8 * atto, jnp.zeros((15,), jnp.float32)])
    acc3 = _edge_phase(1)(X3[:, :32], X3[:, 32:64], src, dst,
                          att3, jnp.zeros((RPT, 32), jnp.float32))

    bc = jnp.broadcast_to((bo + blino)[None, :], (1, 16))
    out16 = _final(acc3, X3[:, 64:80], bc)
    return out16[:, 0:1]


# speedup vs baseline: 3.4869x; 1.0438x over previous
import functools

import jax
import jax.numpy as jnp
from jax import lax
from jax.experimental import pallas as pl
from jax.experimental.pallas import tpu as pltpu
from jax.experimental.pallas import tpu_sc as plsc

N = 10000
E = 320000
NC, NS, L = 2, 16, 16
NW = NC * NS
EW = E // NW
B = 64
NCHUNK = EW // B
TAIL = EW - NCHUNK * B
NG = B // L
RPT = N // NS


@functools.lru_cache(maxsize=None)
def _edge_phase(KG):
    D = KG * L
    DM = D + L
    mesh = plsc.VectorSubcoreMesh(core_axis_name="c", subcore_axis_name="s",
                                  num_cores=NC, num_subcores=NS)

    @functools.partial(
        pl.kernel,
        mesh=mesh,
        compiler_params=pltpu.CompilerParams(use_tc_tiling_on_sc=False,
                                             needs_layout_passes=False),
        out_type=jax.ShapeDtypeStruct((NC, N, DM), jnp.float32),
        scratch_types=[
            pltpu.VMEM((4, B), jnp.int32),
            pltpu.VMEM((4, B), jnp.int32),
            pltpu.VMEM((B, DM), jnp.float32),
            pltpu.VMEM((B, DM), jnp.float32),
            pltpu.VMEM((B, DM), jnp.float32),
            pltpu.VMEM((B, DM), jnp.float32),
            pltpu.VMEM((D,), jnp.float32),
            pltpu.VMEM((TAIL,), jnp.int32),
            pltpu.VMEM((TAIL,), jnp.int32),
            pltpu.VMEM_SHARED((N, DM), jnp.float32),
            pltpu.SemaphoreType.DMA,
            pltpu.SemaphoreType.DMA,
            pltpu.SemaphoreType.DMA,
            pltpu.SemaphoreType.DMA,
            pltpu.SemaphoreType.DMA,
            pltpu.SemaphoreType.DMA,
            pltpu.SemaphoreType.DMA,
            pltpu.SemaphoreType.DMA,
        ],
    )
    def edge_kernel(xl_hbm, xr_hbm, src_hbm, dst_hbm, att_hbm, zer_hbm,
                    acc_out,
                    src_i, dst_i, xl0, xl1, xr0, xr1, att_v, src_t, dst_t,
                    acc_sh, i0, i1, i2, i3, g0, g1, s0, s1):
        cid = lax.axis_index("c")
        sid = lax.axis_index("s")
        wid = sid * NC + cid
        xl_r, xr_r = [xl0, xl1], [xr0, xr1]
        isem, gsem, ssem = [i0, i1, i2, i3], [g0, g1], [s0, s1]

        row0 = sid * RPT
        pltpu.sync_copy(zer_hbm, acc_sh.at[pl.ds(row0, RPT), :])
        pltpu.sync_copy(att_hbm, att_v)
        plsc.subcore_barrier()

        base = wid * EW
        last = NCHUNK - 1
        att_k = [att_v[pl.ds(k * L, L)] for k in range(KG)]
        lane0 = (lax.iota(jnp.int32, L) == 0).astype(jnp.float32)

        def idx_fetch(chunk, slot, sem):
            off = base + chunk * B
            pltpu.make_async_copy(src_hbm.at[pl.ds(off, B)],
                                  src_i.at[slot], sem).start()
            pltpu.make_async_copy(dst_hbm.at[pl.ds(off, B)],
                                  dst_i.at[slot], sem).start()

        def idx_wait(slot, sem):
            pltpu.make_async_copy(src_hbm.at[pl.ds(base, B)],
                                  src_i.at[slot], sem).wait()
            pltpu.make_async_copy(dst_hbm.at[pl.ds(base, B)],
                                  dst_i.at[slot], sem).wait()

        def gather_start(slot, p):
            pltpu.make_async_copy(xl_hbm.at[src_i.at[slot]],
                                  xl_r[p], gsem[p]).start()
            pltpu.make_async_copy(xr_hbm.at[dst_i.at[slot]],
                                  xr_r[p], gsem[p]).start()

        def gather_wait(slot, p):
            pltpu.make_async_copy(xl_hbm.at[src_i.at[slot]],
                                  xl_r[p], gsem[p]).wait()
            pltpu.make_async_copy(xr_hbm.at[dst_i.at[slot]],
                                  xr_r[p], gsem[p]).wait()

        def scatter_start(slot, p):
            pltpu.make_async_copy(xl_r[p], acc_sh.at[dst_i.at[slot]],
                                  ssem[p]).start(add=True)

        def scatter_wait(p):
            pltpu.make_async_copy(xl_r[p], acc_sh.at[dst_i.at[0]],
                                  ssem[p]).wait()

        def compute(xlb, xrb, ne):
            def one(e):
                acc = None
                xls = []
                for k in range(KG):
                    a = xlb[e, pl.ds(k * L, L)]
                    b = xrb[e, pl.ds(k * L, L)]
                    t = jnp.maximum(a + b, 0.0) * att_k[k]
                    acc = t if acc is None else acc + t
                    xls.append(a)
                acc = acc + xlb[e, pl.ds(D, L)] + xrb[e, pl.ds(D, L)]
                wv = jnp.exp(jnp.full((L,), jnp.sum(acc), jnp.float32))
                xlb[e, pl.ds(D, L)] = wv * lane0
                for k in range(KG):
                    xlb[e, pl.ds(k * L, L)] = xls[k] * wv

            def four(j, _):
                for q in range(4):
                    one(4 * j + q)
                return 0
            lax.fori_loop(0, ne // 4, four, 0)

        def process(c, s_idx):
            p = s_idx % 2
            s_nxt, s_pre = (s_idx + 1) % 4, (s_idx + 2) % 4
            p_nxt = (p + 1) % 2
            gather_wait(s_idx, p)
            idx_fetch(jnp.minimum(c + 2, last), s_pre, isem[s_pre])

            @pl.when(c >= 1)
            def _():
                scatter_wait(p_nxt)
            idx_wait(s_nxt, isem[s_nxt])
            gather_start(s_nxt, p_nxt)
            compute(xl_r[p], xr_r[p], B)
            scatter_start(s_idx, p)

        pltpu.sync_copy(src_hbm.at[pl.ds(base, B)], src_i.at[0])
        pltpu.sync_copy(dst_hbm.at[pl.ds(base, B)], dst_i.at[0])
        idx_fetch(1, 1, isem[1])
        gather_start(0, 0)

        def quad(i, _):
            for c4 in range(4):
                process(4 * i + c4, c4)
            return 0
        lax.fori_loop(0, NCHUNK // 4, quad, 0)
        for c in range(4 * (NCHUNK // 4), NCHUNK):
            process(c, c % 4)

        idx_wait((last + 2) % 4, isem[(last + 2) % 4])
        gather_wait((last + 1) % 4, (last + 1) % 2)
        scatter_wait(last % 2)

        toff = base + NCHUNK * B
        pltpu.sync_copy(src_hbm.at[pl.ds(toff, TAIL)], src_t)
        pltpu.sync_copy(dst_hbm.at[pl.ds(toff, TAIL)], dst_t)
        pltpu.sync_copy(xl_hbm.at[src_t], xl0.at[pl.ds(0, TAIL), :])
        pltpu.sync_copy(xr_hbm.at[dst_t], xr0.at[pl.ds(0, TAIL), :])
        compute(xl0, xr0, TAIL)
        pltpu.sync_copy(xl0.at[pl.ds(0, TAIL), :], acc_sh.at[dst_t],
                        add=True)

        plsc.subcore_barrier()
        pltpu.sync_copy(acc_sh.at[pl.ds(row0, RPT), :],
                        acc_out.at[cid, pl.ds(row0, RPT), :])

    return edge_kernel


def _matmul3(x, wl, wr, ws):
    R = 1000
    K = x.shape[1]

    def body(x_ref, wl_ref, wr_ref, ws_ref, a_ref, b_ref, c_ref):
        xv = x_ref[...]
        a_ref[...] = jnp.dot(xv, wl_ref[...],
                             preferred_element_type=jnp.float32)
        b_ref[...] = jnp.dot(xv, wr_ref[...],
                             preferred_element_type=jnp.float32)
        c_ref[...] = jnp.dot(xv, ws_ref[...],
                             preferred_element_type=jnp.float32)

    return pl.pallas_call(
        body,
        grid=(N // R,),
        in_specs=[pl.BlockSpec((R, K), lambda i: (i, 0))] +
                 [pl.BlockSpec(w.shape, lambda i: (0, 0))
                  for w in (wl, wr, ws)],
        out_specs=[pl.BlockSpec((R, w.shape[1]), lambda i: (i, 0))
                   for w in (wl, wr, ws)],
        out_shape=[jax.ShapeDtypeStruct((N, w.shape[1]), jnp.float32)
                   for w in (wl, wr, ws)],
    )(x, wl, wr, ws)


def _combine_matmul3(acc, skip, bias, wl, wr, ws):
    R = 1000
    D = acc.shape[2] - 16

    def body(a_ref, s_ref, b_ref, wl_ref, wr_ref, ws_ref,
             o1_ref, o2_ref, o3_ref):
        raw = a_ref[0] + a_ref[1]
        ns = raw[:, :D]
        den = raw[:, D:D + 1]
        h = ns / (den + 1e-16) + s_ref[...] + b_ref[...]
        h = jnp.maximum(h, 0.0)
        o1_ref[...] = jnp.dot(h, wl_ref[...],
                              preferred_element_type=jnp.float32)
        o2_ref[...] = jnp.dot(h, wr_ref[...],
                              preferred_element_type=jnp.float32)
        o3_ref[...] = jnp.dot(h, ws_ref[...],
                              preferred_element_type=jnp.float32)

    return pl.pallas_call(
        body,
        grid=(N // R,),
        in_specs=[pl.BlockSpec((NC, R, D + 16), lambda i: (0, i, 0)),
                  pl.BlockSpec((R, D), lambda i: (i, 0)),
                  pl.BlockSpec((1, D), lambda i: (0, 0))] +
                 [pl.BlockSpec(w.shape, lambda i: (0, 0))
                  for w in (wl, wr, ws)],
        out_specs=[pl.BlockSpec((R, w.shape[1]), lambda i: (i, 0))
                   for w in (wl, wr, ws)],
        out_shape=[jax.ShapeDtypeStruct((N, w.shape[1]), jnp.float32)
                   for w in (wl, wr, ws)],
    )(acc, skip, bias, wl, wr, ws)


def _final(acc, skip, bias):
    R = 1000

    def body(a_ref, s_ref, b_ref, o_ref):
        raw = a_ref[0] + a_ref[1]
        num = raw[:, 0:1]
        den = raw[:, 16:17]
        o_ref[...] = num / (den + 1e-16) + s_ref[:, 0:1] + b_ref[...]

    return pl.pallas_call(
        body,
        grid=(N // R,),
        in_specs=[pl.BlockSpec((NC, R, 32), lambda i: (0, i, 0)),
                  pl.BlockSpec((R, 16), lambda i: (i, 0)),
                  pl.BlockSpec((1, 16), lambda i: (0, 0))],
        out_specs=pl.BlockSpec((R, 16), lambda i: (i, 0)),
        out_shape=jax.ShapeDtypeStruct((N, 16), jnp.float32),
    )(acc, skip, bias)


def kernel(x, edge_index, Wl1, Wr1, att1, b1, Wlin1, blin1,
           Wl2, Wr2, att2, b2, Wlin2, blin2,
           Wlo, Wro, atto, bo, Wlino, blino):
    src = edge_index[0]
    dst = edge_index[1]

    zer144 = jnp.zeros((RPT, 144), jnp.float32)
    z15 = jnp.zeros((128, 15), jnp.float32)

    def wside(W, att):
        return jnp.concatenate([W, 0.2 * (W @ att)[:, None], z15], axis=1)

    XL1, XR1, SK1 = _matmul3(x, wside(Wl1, att1), wside(Wr1, att1), Wlin1)
    acc1 = _edge_phase(8)(XL1, XR1, src, dst, 0.8 * att1, zer144)

    XL2, XR2, SK2 = _combine_matmul3(
        acc1, SK1, (b1 + blin1)[None, :],
        wside(Wl2, att2), wside(Wr2, att2), Wlin2)
    acc2 = _edge_phase(8)(XL2, XR2, src, dst, 0.8 * att2, zer144)

    def wside3(W):
        return jnp.concatenate([W, z15, 0.2 * atto[0] * W, z15], axis=1)

    XL3, XR3, SK3 = _combine_matmul3(
        acc2, SK2, (b2 + blin2)[None, :],
        wside3(Wlo), wside3(Wro),
        jnp.concatenate([Wlino, z15], axis=1))

    att3 = jnp.concatenate([0.8 * atto, jnp.zeros((15,), jnp.float32)])
    acc3 = _edge_phase(1)(XL3, XR3, src, dst,
                          att3, jnp.zeros((RPT, 32), jnp.float32))

    bc = jnp.broadcast_to((bo + blino)[None, :], (1, 16))
    out16 = _final(acc3, SK3, bc)
    return out16[:, 0:1]


# speedup vs baseline: 3.6548x; 1.0481x over previous
import functools

import jax
import jax.numpy as jnp
from jax import lax
from jax.experimental import pallas as pl
from jax.experimental.pallas import tpu as pltpu
from jax.experimental.pallas import tpu_sc as plsc

N = 10000
E = 320000
NC, NS, L = 2, 16, 16
NW = NC * NS
EW = E // NW
RPT = N // NS


@functools.lru_cache(maxsize=None)
def _edge_phase(KG):
    D = KG * L
    DM = D + L
    B = 64 if KG == 8 else 128
    NCHUNK = EW // B
    TAIL = EW - NCHUNK * B
    mesh = plsc.VectorSubcoreMesh(core_axis_name="c", subcore_axis_name="s",
                                  num_cores=NC, num_subcores=NS)

    @functools.partial(
        pl.kernel,
        mesh=mesh,
        compiler_params=pltpu.CompilerParams(use_tc_tiling_on_sc=False,
                                             needs_layout_passes=False),
        out_type=jax.ShapeDtypeStruct((NC, N, DM), jnp.float32),
        scratch_types=[
            pltpu.VMEM((4, B), jnp.int32),
            pltpu.VMEM((4, B), jnp.int32),
            pltpu.VMEM((B, DM), jnp.float32),
            pltpu.VMEM((B, DM), jnp.float32),
            pltpu.VMEM((B, DM), jnp.float32),
            pltpu.VMEM((B, DM), jnp.float32),
            pltpu.VMEM((D,), jnp.float32),
            pltpu.VMEM((TAIL,), jnp.int32),
            pltpu.VMEM((TAIL,), jnp.int32),
            pltpu.VMEM_SHARED((N, DM), jnp.float32),
            pltpu.SemaphoreType.DMA,
            pltpu.SemaphoreType.DMA,
            pltpu.SemaphoreType.DMA,
            pltpu.SemaphoreType.DMA,
            pltpu.SemaphoreType.DMA,
            pltpu.SemaphoreType.DMA,
            pltpu.SemaphoreType.DMA,
            pltpu.SemaphoreType.DMA,
        ],
    )
    def edge_kernel(xl_hbm, xr_hbm, src_hbm, dst_hbm, att_hbm, zer_hbm,
                    acc_out,
                    src_i, dst_i, xl0, xl1, xr0, xr1, att_v, src_t, dst_t,
                    acc_sh, i0, i1, i2, i3, g0, g1, s0, s1):
        cid = lax.axis_index("c")
        sid = lax.axis_index("s")
        wid = sid * NC + cid
        xl_r, xr_r = [xl0, xl1], [xr0, xr1]
        isem, gsem, ssem = [i0, i1, i2, i3], [g0, g1], [s0, s1]

        row0 = sid * RPT
        pltpu.sync_copy(zer_hbm, acc_sh.at[pl.ds(row0, RPT), :])
        pltpu.sync_copy(att_hbm, att_v)
        plsc.subcore_barrier()

        base = wid * EW
        last = NCHUNK - 1
        att_k = [att_v[pl.ds(k * L, L)] for k in range(KG)]
        lane0 = (lax.iota(jnp.int32, L) == 0).astype(jnp.float32)

        def idx_fetch(chunk, slot, sem):
            off = base + chunk * B
            pltpu.make_async_copy(src_hbm.at[pl.ds(off, B)],
                                  src_i.at[slot], sem).start()
            pltpu.make_async_copy(dst_hbm.at[pl.ds(off, B)],
                                  dst_i.at[slot], sem).start()

        def idx_wait(slot, sem):
            pltpu.make_async_copy(src_hbm.at[pl.ds(base, B)],
                                  src_i.at[slot], sem).wait()
            pltpu.make_async_copy(dst_hbm.at[pl.ds(base, B)],
                                  dst_i.at[slot], sem).wait()

        def gather_start(slot, p):
            pltpu.make_async_copy(xl_hbm.at[src_i.at[slot]],
                                  xl_r[p], gsem[p]).start()
            pltpu.make_async_copy(xr_hbm.at[dst_i.at[slot]],
                                  xr_r[p], gsem[p]).start()

        def gather_wait(slot, p):
            pltpu.make_async_copy(xl_hbm.at[src_i.at[slot]],
                                  xl_r[p], gsem[p]).wait()
            pltpu.make_async_copy(xr_hbm.at[dst_i.at[slot]],
                                  xr_r[p], gsem[p]).wait()

        def scatter_start(slot, p):
            pltpu.make_async_copy(xl_r[p], acc_sh.at[dst_i.at[slot]],
                                  ssem[p]).start(add=True)

        def scatter_wait(p):
            pltpu.make_async_copy(xl_r[p], acc_sh.at[dst_i.at[0]],
                                  ssem[p]).wait()

        def compute(xlb, xrb, ne):
            def one(e):
                acc = None
                xls = []
                for k in range(KG):
                    a = xlb[e, pl.ds(k * L, L)]
                    b = xrb[e, pl.ds(k * L, L)]
                    t = jnp.maximum(a + b, 0.0) * att_k[k]
                    acc = t if acc is None else acc + t
                    xls.append(a)
                acc = acc + xlb[e, pl.ds(D, L)] + xrb[e, pl.ds(D, L)]
                wv = jnp.exp(jnp.full((L,), jnp.sum(acc), jnp.float32))
                xlb[e, pl.ds(D, L)] = wv * lane0
                for k in range(KG):
                    xlb[e, pl.ds(k * L, L)] = xls[k] * wv

            def four(j, _):
                for q in range(4):
                    one(4 * j + q)
                return 0
            lax.fori_loop(0, ne // 4, four, 0)

        def process(c, s_idx):
            p = s_idx % 2
            s_nxt, s_pre = (s_idx + 1) % 4, (s_idx + 2) % 4
            p_nxt = (p + 1) % 2
            gather_wait(s_idx, p)
            idx_fetch(jnp.minimum(c + 2, last), s_pre, isem[s_pre])

            @pl.when(c >= 1)
            def _():
                scatter_wait(p_nxt)
            idx_wait(s_nxt, isem[s_nxt])
            gather_start(s_nxt, p_nxt)
            compute(xl_r[p], xr_r[p], B)
            scatter_start(s_idx, p)

        pltpu.sync_copy(src_hbm.at[pl.ds(base, B)], src_i.at[0])
        pltpu.sync_copy(dst_hbm.at[pl.ds(base, B)], dst_i.at[0])
        idx_fetch(1, 1, isem[1])
        gather_start(0, 0)

        def quad(i, _):
            for c4 in range(4):
                process(4 * i + c4, c4)
            return 0
        lax.fori_loop(0, NCHUNK // 4, quad, 0)
        for c in range(4 * (NCHUNK // 4), NCHUNK):
            process(c, c % 4)

        idx_wait((last + 2) % 4, isem[(last + 2) % 4])
        gather_wait((last + 1) % 4, (last + 1) % 2)
        scatter_wait(last % 2)

        toff = base + NCHUNK * B
        pltpu.sync_copy(src_hbm.at[pl.ds(toff, TAIL)], src_t)
        pltpu.sync_copy(dst_hbm.at[pl.ds(toff, TAIL)], dst_t)
        pltpu.sync_copy(xl_hbm.at[src_t], xl0.at[pl.ds(0, TAIL), :])
        pltpu.sync_copy(xr_hbm.at[dst_t], xr0.at[pl.ds(0, TAIL), :])
        compute(xl0, xr0, TAIL)
        pltpu.sync_copy(xl0.at[pl.ds(0, TAIL), :], acc_sh.at[dst_t],
                        add=True)

        plsc.subcore_barrier()
        pltpu.sync_copy(acc_sh.at[pl.ds(row0, RPT), :],
                        acc_out.at[cid, pl.ds(row0, RPT), :])

    return edge_kernel


def _matmul3(x, wl, wr, ws):
    R = 1000
    K = x.shape[1]

    def body(x_ref, wl_ref, wr_ref, ws_ref, a_ref, b_ref, c_ref):
        xv = x_ref[...]
        a_ref[...] = jnp.dot(xv, wl_ref[...],
                             preferred_element_type=jnp.float32)
        b_ref[...] = jnp.dot(xv, wr_ref[...],
                             preferred_element_type=jnp.float32)
        c_ref[...] = jnp.dot(xv, ws_ref[...],
                             preferred_element_type=jnp.float32)

    return pl.pallas_call(
        body,
        grid=(N // R,),
        in_specs=[pl.BlockSpec((R, K), lambda i: (i, 0))] +
                 [pl.BlockSpec(w.shape, lambda i: (0, 0))
                  for w in (wl, wr, ws)],
        out_specs=[pl.BlockSpec((R, w.shape[1]), lambda i: (i, 0))
                   for w in (wl, wr, ws)],
        out_shape=[jax.ShapeDtypeStruct((N, w.shape[1]), jnp.float32)
                   for w in (wl, wr, ws)],
    )(x, wl, wr, ws)


def _combine_matmul3(acc, skip, bias, wl, wr, ws):
    R = 1000
    D = acc.shape[2] - 16

    def body(a_ref, s_ref, b_ref, wl_ref, wr_ref, ws_ref,
             o1_ref, o2_ref, o3_ref):
        raw = a_ref[0] + a_ref[1]
        ns = raw[:, :D]
        den = raw[:, D:D + 1]
        h = ns / (den + 1e-16) + s_ref[...] + b_ref[...]
        h = jnp.maximum(h, 0.0)
        o1_ref[...] = jnp.dot(h, wl_ref[...],
                              preferred_element_type=jnp.float32)
        o2_ref[...] = jnp.dot(h, wr_ref[...],
                              preferred_element_type=jnp.float32)
        o3_ref[...] = jnp.dot(h, ws_ref[...],
                              preferred_element_type=jnp.float32)

    return pl.pallas_call(
        body,
        grid=(N // R,),
        in_specs=[pl.BlockSpec((NC, R, D + 16), lambda i: (0, i, 0)),
                  pl.BlockSpec((R, D), lambda i: (i, 0)),
                  pl.BlockSpec((1, D), lambda i: (0, 0))] +
                 [pl.BlockSpec(w.shape, lambda i: (0, 0))
                  for w in (wl, wr, ws)],
        out_specs=[pl.BlockSpec((R, w.shape[1]), lambda i: (i, 0))
                   for w in (wl, wr, ws)],
        out_shape=[jax.ShapeDtypeStruct((N, w.shape[1]), jnp.float32)
                   for w in (wl, wr, ws)],
    )(acc, skip, bias, wl, wr, ws)


def _final(acc, skip, bias):
    R = 1000

    def body(a_ref, s_ref, b_ref, o_ref):
        raw = a_ref[0] + a_ref[1]
        num = raw[:, 0:1]
        den = raw[:, 16:17]
        o_ref[...] = num / (den + 1e-16) + s_ref[:, 0:1] + b_ref[...]

    return pl.pallas_call(
        body,
        grid=(N // R,),
        in_specs=[pl.BlockSpec((NC, R, 32), lambda i: (0, i, 0)),
                  pl.BlockSpec((R, 16), lambda i: (i, 0)),
                  pl.BlockSpec((1, 16), lambda i: (0, 0))],
        out_specs=pl.BlockSpec((R, 16), lambda i: (i, 0)),
        out_shape=jax.ShapeDtypeStruct((N, 16), jnp.float32),
    )(acc, skip, bias)


def kernel(x, edge_index, Wl1, Wr1, att1, b1, Wlin1, blin1,
           Wl2, Wr2, att2, b2, Wlin2, blin2,
           Wlo, Wro, atto, bo, Wlino, blino):
    src = edge_index[0]
    dst = edge_index[1]

    zer144 = jnp.zeros((RPT, 144), jnp.float32)
    z15 = jnp.zeros((128, 15), jnp.float32)

    def wside(W, att):
        return jnp.concatenate([W, 0.2 * (W @ att)[:, None], z15], axis=1)

    XL1, XR1, SK1 = _matmul3(x, wside(Wl1, att1), wside(Wr1, att1), Wlin1)
    acc1 = _edge_phase(8)(XL1, XR1, src, dst, 0.8 * att1, zer144)

    XL2, XR2, SK2 = _combine_matmul3(
        acc1, SK1, (b1 + blin1)[None, :],
        wside(Wl2, att2), wside(Wr2, att2), Wlin2)
    acc2 = _edge_phase(8)(XL2, XR2, src, dst, 0.8 * att2, zer144)

    def wside3(W):
        return jnp.concatenate([W, z15, 0.2 * atto[0] * W, z15], axis=1)

    XL3, XR3, SK3 = _combine_matmul3(
        acc2, SK2, (b2 + blin2)[None, :],
        wside3(Wlo), wside3(Wro),
        jnp.concatenate([Wlino, z15], axis=1))

    att3 = jnp.concatenate([0.8 * atto, jnp.zeros((15,), jnp.float32)])
    acc3 = _edge_phase(1)(XL3, XR3, src, dst,
                          att3, jnp.zeros((RPT, 32), jnp.float32))

    bc = jnp.broadcast_to((bo + blino)[None, :], (1, 16))
    out16 = _final(acc3, SK3, bc)
    return out16[:, 0:1]
